# baseline (speedup 1.0000x reference)
import jax
import jax.numpy as jnp
from jax.experimental import pallas as pl

_BREAKS = [0, 80000, 160000, 240000, 320000]


def _gcn_conv(x, edge_index, W, b):
    n = x.shape[0]
    loop = jnp.arange(n)
    src = jnp.concatenate([edge_index[0], loop])
    dst = jnp.concatenate([edge_index[1], loop])
    h = x @ W
    deg = jnp.zeros((n,), dtype=x.dtype).at[dst].add(1.0)
    dis = jnp.where(deg > 0, 1.0 / jnp.sqrt(deg), 0.0)
    norm = dis[src] * dis[dst]
    msg = h[src] * norm[:, None]
    out = jnp.zeros((n, h.shape[1]), dtype=x.dtype).at[dst].add(msg)
    return out + b


def _batch_norm(x, g, b):
    m = jnp.mean(x, axis=0)
    v = jnp.var(x, axis=0)
    return (x - m) / jnp.sqrt(v + 1e-5) * g + b


def _final_linear_kernel(xa_ref, xf_ref, xb_ref, w_ref, b_ref, o_ref):
    cat = jnp.concatenate([xa_ref[...], xf_ref[...], xb_ref[...]], axis=1)
    o_ref[...] = cat @ w_ref[...] + b_ref[...][None, :]


def kernel(x, edge_index, lin1_w, lin1_b, conv_w, conv_b, convf_w, convf_b,
           convb_w, convb_b, bn_g, bn_b, bnf_g, bnf_b, bnb_g, bnb_b,
           lin2_w, lin2_b):
    x1 = x @ lin1_w + lin1_b
    x_a = x1
    x_f = x1
    x_b = x1
    for i in range(len(_BREAKS) - 1):
        sub = edge_index[:, _BREAKS[i]:_BREAKS[i + 1]]
        x_f = _batch_norm(_gcn_conv(x_f, sub, convf_w, convf_b), bnf_g, bnf_b)
    for i in range(len(_BREAKS) - 1, 0, -1):
        sub = edge_index[:, _BREAKS[i - 1]:_BREAKS[i]]
        x_b = _batch_norm(_gcn_conv(x_b, sub, convb_w, convb_b), bnb_g, bnb_b)
    x_a = _batch_norm(_gcn_conv(x_a, edge_index, conv_w, conv_b), bn_g, bn_b)
    n = x.shape[0]
    d = lin2_w.shape[1]
    return pl.pallas_call(
        _final_linear_kernel,
        out_shape=jax.ShapeDtypeStruct((n, d), jnp.float32),
    )(x_a, x_f, x_b, lin2_w, lin2_b)


# speedup vs baseline: 26.9139x; 26.9139x over previous
import jax
import jax.numpy as jnp
from jax import lax
from jax.experimental import pallas as pl
from jax.experimental.pallas import tpu as pltpu
from jax.experimental.pallas import tpu_sc as plsc

_N = 10000
_NPAD = 10112
_NC, _NS = 2, 16
_NW = _NC * _NS
_CH = 128
_KWIN = 20
_KFULL = 80
_RPT = _NPAD // _NS
_D = 32

_mesh = plsc.VectorSubcoreMesh(core_axis_name="c", subcore_axis_name="s")
_sc_params = pltpu.CompilerParams(use_tc_tiling_on_sc=False)


def _deg_body(d0, d1, d2, d3, ones_hbm, zeros_hbm, out, idx_v, ones_v, acc):
    c = lax.axis_index("c")
    s = lax.axis_index("s")
    wid = c * _NS + s
    r0 = s * _RPT
    pltpu.sync_copy(ones_hbm, ones_v)
    for w in range(4):
        pltpu.sync_copy(zeros_hbm, acc.at[w, pl.ds(r0, _RPT)])
    plsc.subcore_barrier()
    for w, dref in enumerate((d0, d1, d2, d3)):
        pltpu.sync_copy(dref.at[wid], idx_v)

        def _one(j, carry, _w=w):
            pltpu.sync_copy(ones_v, acc.at[_w].at[idx_v.at[j]], add=True)
            return carry

        lax.fori_loop(0, _KWIN, _one, 0)
    plsc.subcore_barrier()
    for w in range(4):
        pltpu.sync_copy(acc.at[w, pl.ds(r0, _RPT)],
                        out.at[c, w, pl.ds(r0, _RPT)])


_deg_call = pl.kernel(
    _deg_body,
    out_type=jax.ShapeDtypeStruct((_NC, 4, _NPAD), jnp.float32),
    mesh=_mesh,
    compiler_params=_sc_params,
    scratch_types=[
        pltpu.VMEM((_KWIN, _CH), jnp.int32),
        pltpu.VMEM((_CH,), jnp.float32),
        pltpu.VMEM_SHARED((4, _NPAD), jnp.float32),
    ],
)


def _make_conv_call(Ks):
    T = len(Ks)

    def body(*refs):
        srcs = [refs[3 * t] for t in range(T)]
        dsts = [refs[3 * t + 1] for t in range(T)]
        hps = [refs[3 * t + 2] for t in range(T)]
        zeros_hbm = refs[3 * T]
        out = refs[3 * T + 1]
        sp = 3 * T + 2
        idxs = refs[sp: sp + 2 * T]
        rows0, rows1 = refs[sp + 2 * T], refs[sp + 2 * T + 1]
        sem0, sem1 = refs[sp + 2 * T + 2], refs[sp + 2 * T + 3]
        accs = refs[sp + 2 * T + 4: sp + 3 * T + 4]

        c = lax.axis_index("c")
        s = lax.axis_index("s")
        wid = c * _NS + s
        r0 = s * _RPT

        for t in range(T):
            @pl.when(c == 0)
            def _(t=t):
                pltpu.sync_copy(hps[t].at[pl.ds(r0, _RPT)],
                                accs[t].at[pl.ds(r0, _RPT)])

            @pl.when(c != 0)
            def _(t=t):
                pltpu.sync_copy(zeros_hbm, accs[t].at[pl.ds(r0, _RPT)])
        plsc.subcore_barrier()

        for t, K in enumerate(Ks):
            isv, idv = idxs[2 * t], idxs[2 * t + 1]
            pltpu.sync_copy(srcs[t].at[wid], isv)
            pltpu.sync_copy(dsts[t].at[wid], idv)
            hp, acc = hps[t], accs[t]
            pltpu.async_copy(hp.at[isv.at[0]], rows0, sem0)
            pltpu.async_copy(hp.at[isv.at[1]], rows1, sem1)

            def _pair(i, carry, _hp=hp, _acc=acc, _isv=isv, _idv=idv, _K=K):
                j0 = 2 * i
                pltpu.make_async_copy(_hp.at[_isv.at[j0]], rows0, sem0).wait()
                pltpu.sync_copy(rows0, _acc.at[_idv.at[j0]], add=True)

                @pl.when(j0 + 2 < _K)
                def _():
                    pltpu.async_copy(_hp.at[_isv.at[j0 + 2]], rows0, sem0)

                pltpu.make_async_copy(_hp.at[_isv.at[j0 + 1]], rows1, sem1).wait()
                pltpu.sync_copy(rows1, _acc.at[_idv.at[j0 + 1]], add=True)

                @pl.when(j0 + 3 < _K)
                def _():
                    pltpu.async_copy(_hp.at[_isv.at[j0 + 3]], rows1, sem1)

                return carry

            lax.fori_loop(0, K // 2, _pair, 0)

        plsc.subcore_barrier()
        for t in range(T):
            pltpu.sync_copy(accs[t].at[pl.ds(r0, _RPT)],
                            out.at[c, pl.ds(r0, _RPT), pl.ds(t * _D, _D)])

    scratch = []
    for K in Ks:
        scratch.append(pltpu.VMEM((K, _CH), jnp.int32))
        scratch.append(pltpu.VMEM((K, _CH), jnp.int32))
    scratch += [
        pltpu.VMEM((_CH, _D), jnp.float32),
        pltpu.VMEM((_CH, _D), jnp.float32),
        pltpu.SemaphoreType.DMA,
        pltpu.SemaphoreType.DMA,
    ]
    scratch += [pltpu.VMEM_SHARED((_NPAD, _D), jnp.float32) for _ in range(T)]

    return pl.kernel(
        body,
        out_type=jax.ShapeDtypeStruct((_NC, _NPAD, _D * T), jnp.float32),
        mesh=_mesh,
        compiler_params=_sc_params,
        scratch_types=scratch,
    )


_conv3 = _make_conv_call([_KFULL, _KWIN, _KWIN])
_conv2 = _make_conv_call([_KWIN, _KWIN])


def _epi(accsum, dis_col, bias, g, bvec):
    y = dis_col * accsum + bias[None, :]
    yv = y[:_N]
    m = jnp.mean(yv, axis=0)
    var = jnp.mean(yv * yv, axis=0) - m * m
    scale = lax.rsqrt(var + 1e-5) * g
    return (y - m[None, :]) * scale[None, :] + bvec[None, :]


def _tc0_body(xp, l1w, l1b, cw, cfw, cbw, degp,
              hp_a, hp_f1, hp_b4, dis8):
    dsum = degp[0] + degp[1]
    degf = dsum[0:1] + dsum[1:2] + dsum[2:3] + dsum[3:4] - 3.0
    dis = lax.rsqrt(jnp.concatenate(
        [dsum, degf, jnp.ones((3, _NPAD), jnp.float32)], axis=0))
    d8 = dis.T
    dis8[...] = d8
    x1 = xp[...] @ l1w[...] + l1b[...][None, :]
    hp_a[...] = d8[:, 4:5] * (x1 @ cw[...])
    hp_f1[...] = d8[:, 0:1] * (x1 @ cfw[...])
    hp_b4[...] = d8[:, 3:4] * (x1 @ cbw[...])


_hp_t = jax.ShapeDtypeStruct((_NPAD, _D), jnp.float32)

_tc0 = pl.pallas_call(
    _tc0_body,
    out_shape=(_hp_t, _hp_t, _hp_t,
               jax.ShapeDtypeStruct((_NPAD, 8), jnp.float32)),
)


def _tc1_body(acc, dis8, conv_b, bn_g, bn_b, convf_b, bnf_g, bnf_b,
              convb_b, bnb_g, bnb_b, cfw, cbw,
              xa_o, hp_f2, hp_b3):
    d8 = dis8[...]
    asum = acc[0] + acc[1]
    xa_o[...] = _epi(asum[:, 0:32], d8[:, 4:5], conv_b[...], bn_g[...], bn_b[...])
    xf = _epi(asum[:, 32:64], d8[:, 0:1], convf_b[...], bnf_g[...], bnf_b[...])
    hp_f2[...] = d8[:, 1:2] * (xf @ cfw[...])
    xb = _epi(asum[:, 64:96], d8[:, 3:4], convb_b[...], bnb_g[...], bnb_b[...])
    hp_b3[...] = d8[:, 2:3] * (xb @ cbw[...])


_tc1 = pl.pallas_call(_tc1_body, out_shape=(_hp_t, _hp_t, _hp_t))


def _make_tc_mid(fcol, bcol, fcol_next, bcol_next):
    def body(acc, dis8, convf_b, bnf_g, bnf_b, convb_b, bnb_g, bnb_b,
             cfw, cbw, hp_f_n, hp_b_n):
        d8 = dis8[...]
        asum = acc[0] + acc[1]
        xf = _epi(asum[:, 0:32], d8[:, fcol:fcol + 1],
                  convf_b[...], bnf_g[...], bnf_b[...])
        hp_f_n[...] = d8[:, fcol_next:fcol_next + 1] * (xf @ cfw[...])
        xb = _epi(asum[:, 32:64], d8[:, bcol:bcol + 1],
                  convb_b[...], bnb_g[...], bnb_b[...])
        hp_b_n[...] = d8[:, bcol_next:bcol_next + 1] * (xb @ cbw[...])

    return pl.pallas_call(body, out_shape=(_hp_t, _hp_t))


_tc2 = _make_tc_mid(1, 2, 2, 1)
_tc3 = _make_tc_mid(2, 1, 3, 0)


def _tc4_body(acc, dis8, convf_b, bnf_g, bnf_b, convb_b, bnb_g, bnb_b,
              xa, l2w, l2b, out):
    d8 = dis8[...]
    asum = acc[0] + acc[1]
    xf = _epi(asum[:, 0:32], d8[:, 3:4], convf_b[...], bnf_g[...], bnf_b[...])
    xb = _epi(asum[:, 32:64], d8[:, 0:1], convb_b[...], bnb_g[...], bnb_b[...])
    cat = jnp.concatenate([xa[...][:_N], xf[:_N], xb[:_N]], axis=1)
    out[...] = cat @ l2w[...] + l2b[...][None, :]


_tc4 = pl.pallas_call(
    _tc4_body, out_shape=jax.ShapeDtypeStruct((_N, _D), jnp.float32))


def _pad_split(a, K):
    tot = _NW * K * _CH
    pad = jnp.full((tot - a.shape[0],), _N, jnp.int32)
    return jnp.concatenate([a.astype(jnp.int32), pad]).reshape(_NW, K, _CH)


def kernel(x, edge_index, lin1_w, lin1_b, conv_w, conv_b, convf_w, convf_b,
           convb_w, convb_b, bn_g, bn_b, bnf_g, bnf_b, bnb_g, bnb_b,
           lin2_w, lin2_b):
    ei = edge_index.astype(jnp.int32)
    src_w = [_pad_split(ei[0, w * 80000:(w + 1) * 80000], _KWIN) for w in range(4)]
    dst_w = [_pad_split(ei[1, w * 80000:(w + 1) * 80000], _KWIN) for w in range(4)]
    src_f = _pad_split(ei[0], _KFULL)
    dst_f = _pad_split(ei[1], _KFULL)

    zeros32 = jnp.zeros((_RPT, _D), jnp.float32)
    zeros1 = jnp.zeros((_RPT,), jnp.float32)
    ones1 = jnp.ones((_CH,), jnp.float32)
    xp = jnp.concatenate([x, jnp.zeros((_NPAD - _N, x.shape[1]), x.dtype)], axis=0)

    degp = _deg_call(dst_w[0], dst_w[1], dst_w[2], dst_w[3], ones1, zeros1)

    hp_a, hp_f1, hp_b4, dis8 = _tc0(xp, lin1_w, lin1_b, conv_w, convf_w,
                                    convb_w, degp)

    acc1 = _conv3(src_f, dst_f, hp_a,
                  src_w[0], dst_w[0], hp_f1,
                  src_w[3], dst_w[3], hp_b4, zeros32)

    xa, hp_f2, hp_b3 = _tc1(acc1, dis8,
                            conv_b, bn_g, bn_b, convf_b, bnf_g, bnf_b,
                            convb_b, bnb_g, bnb_b, convf_w, convb_w)

    acc2 = _conv2(src_w[1], dst_w[1], hp_f2,
                  src_w[2], dst_w[2], hp_b3, zeros32)

    hp_f3, hp_b2 = _tc2(acc2, dis8,
                        convf_b, bnf_g, bnf_b, convb_b, bnb_g, bnb_b,
                        convf_w, convb_w)

    acc3 = _conv2(src_w[2], dst_w[2], hp_f3,
                  src_w[1], dst_w[1], hp_b2, zeros32)

    hp_f4, hp_b1 = _tc3(acc3, dis8,
                        convf_b, bnf_g, bnf_b, convb_b, bnb_g, bnb_b,
                        convf_w, convb_w)

    acc4 = _conv2(src_w[3], dst_w[3], hp_f4,
                  src_w[0], dst_w[0], hp_b1, zeros32)

    return _tc4(acc4, dis8,
                convf_b, bnf_g, bnf_b, convb_b, bnb_g, bnb_b,
                xa, lin2_w, lin2_b)
